# speedup vs baseline: 1.4950x; 1.4950x over previous
import functools

import jax
import jax.numpy as jnp
from jax import lax
from jax.experimental import pallas as pl
from jax.experimental.pallas import tpu as pltpu
from jax.experimental.pallas import tpu_sc as plsc

D = 768
N = 8192
NC, NS = 2, 16
NW = NC * NS
BPW = N // NW
C = 128
NCHUNK = BPW // C


def _gather3(idx, t0, t1, t2):
    mesh = plsc.VectorSubcoreMesh(core_axis_name="c", subcore_axis_name="s")
    out_t = (jax.ShapeDtypeStruct((N, D), jnp.float32),) * 3

    @functools.partial(
        pl.kernel,
        out_type=out_t,
        mesh=mesh,
        scratch_types=[
            pltpu.VMEM((NCHUNK, C), jnp.int32),
            pltpu.VMEM((C, D), jnp.float32),
            pltpu.SemaphoreType.DMA,
        ],
    )
    def k(idx_hbm, T0, T1, T2, O0, O1, O2, idx_v, rows_v, sem):
        wid = lax.axis_index("s") * NC + lax.axis_index("c")
        base = wid * BPW
        pltpu.sync_copy(idx_hbm.at[wid], idx_v)
        for T, O in ((T0, O0), (T1, O1), (T2, O2)):
            for c in range(NCHUNK):
                pltpu.async_copy(T.at[idx_v.at[c]], rows_v, sem).wait()
                pltpu.sync_copy(rows_v, O.at[pl.ds(base + c * C, C)])

    return k(idx, t0, t1, t2)


def kernel(inputs, table0, table1, table2):
    B, S = inputs.shape
    idx = inputs.reshape(NW, NCHUNK, C).astype(jnp.int32)
    o0, o1, o2 = _gather3(idx, table0, table1, table2)
    o0 = o0.reshape(B, S, D)
    o1 = o1.reshape(B, S, D)
    o2 = o2.reshape(B, S, D)
    return (o0, o1, o2, o2, o1, o0)


# speedup vs baseline: 1.5094x; 1.0097x over previous
import functools

import jax
import jax.numpy as jnp
from jax import lax
from jax.experimental import pallas as pl
from jax.experimental.pallas import tpu as pltpu
from jax.experimental.pallas import tpu_sc as plsc

D = 768
N = 8192
NC, NS = 2, 16
NW = NC * NS
BPW = N // NW
C = 64
NCHUNK = BPW // C


def _gather3(idx, t0, t1, t2):
    mesh = plsc.VectorSubcoreMesh(core_axis_name="c", subcore_axis_name="s")
    out_t = (jax.ShapeDtypeStruct((N, D), jnp.float32),) * 3

    @functools.partial(
        pl.kernel,
        out_type=out_t,
        mesh=mesh,
        scratch_types=[
            pltpu.VMEM((NCHUNK, C), jnp.int32),
            pltpu.VMEM((C, D), jnp.float32),
            pltpu.VMEM((C, D), jnp.float32),
            pltpu.SemaphoreType.DMA,
            pltpu.SemaphoreType.DMA,
            pltpu.SemaphoreType.DMA,
            pltpu.SemaphoreType.DMA,
        ],
    )
    def k(idx_hbm, T0, T1, T2, O0, O1, O2, idx_v, rows0, rows1,
          g0, g1, w0, w1):
        wid = lax.axis_index("s") * NC + lax.axis_index("c")
        base = wid * BPW
        pltpu.sync_copy(idx_hbm.at[wid], idx_v)
        tasks = [(T, O, c)
                 for (T, O) in ((T0, O0), (T1, O1), (T2, O2))
                 for c in range(NCHUNK)]
        rows = (rows0, rows1)
        gsem = (g0, g1)
        wsem = (w0, w1)

        def start_gather(i):
            T, _, c = tasks[i]
            b = i % 2
            return pltpu.async_copy(T.at[idx_v.at[c]], rows[b], gsem[b])

        def start_write(i):
            _, O, c = tasks[i]
            b = i % 2
            return pltpu.async_copy(rows[b], O.at[pl.ds(base + c * C, C)],
                                    wsem[b])

        n = len(tasks)
        g = [None, None]
        w = [None, None]
        g[0] = start_gather(0)
        for i in range(n):
            b = i % 2
            if i + 1 < n:
                nb = (i + 1) % 2
                if w[nb] is not None:
                    w[nb].wait()
                    w[nb] = None
                g[nb] = start_gather(i + 1)
            g[b].wait()
            w[b] = start_write(i)
        for h in w:
            if h is not None:
                h.wait()

    return k(idx, t0, t1, t2)


def kernel(inputs, table0, table1, table2):
    B, S = inputs.shape
    idx = inputs.reshape(NW, NCHUNK, C).astype(jnp.int32)
    o0, o1, o2 = _gather3(idx, table0, table1, table2)
    o0 = o0.reshape(B, S, D)
    o1 = o1.reshape(B, S, D)
    o2 = o2.reshape(B, S, D)
    return (o0, o1, o2, o2, o1, o0)


# speedup vs baseline: 1.8539x; 1.2282x over previous
import functools

import jax
import jax.numpy as jnp
from jax import lax
from jax.experimental import pallas as pl
from jax.experimental.pallas import tpu as pltpu
from jax.experimental.pallas import tpu_sc as plsc

D = 768
N = 8192
NC, NS = 2, 16
NW = NC * NS
BPW = N // NW
C = 64
NCHUNK = BPW // C


def _gather6(idx, t0, t1, t2):
    mesh = plsc.VectorSubcoreMesh(core_axis_name="c", subcore_axis_name="s")
    out_t = (jax.ShapeDtypeStruct((N, D), jnp.float32),) * 6

    @functools.partial(
        pl.kernel,
        out_type=out_t,
        mesh=mesh,
        scratch_types=[
            pltpu.VMEM((NCHUNK, C), jnp.int32),
            pltpu.VMEM((C, D), jnp.float32),
            pltpu.VMEM((C, D), jnp.float32),
            pltpu.SemaphoreType.DMA,
            pltpu.SemaphoreType.DMA,
            pltpu.SemaphoreType.DMA,
            pltpu.SemaphoreType.DMA,
        ],
    )
    def k(idx_hbm, T0, T1, T2, O0, O1, O2, O3, O4, O5, idx_v, rows0, rows1,
          g0, g1, w0, w1):
        wid = lax.axis_index("s") * NC + lax.axis_index("c")
        base = wid * BPW
        pltpu.sync_copy(idx_hbm.at[wid], idx_v)
        tasks = [(T, Oa, Ob, c)
                 for (T, Oa, Ob) in ((T0, O0, O5), (T1, O1, O4), (T2, O2, O3))
                 for c in range(NCHUNK)]
        rows = (rows0, rows1)
        gsem = (g0, g1)
        wsem = (w0, w1)

        def start_gather(i):
            T, _, _, c = tasks[i]
            b = i % 2
            return pltpu.async_copy(T.at[idx_v.at[c]], rows[b], gsem[b])

        def start_writes(i):
            _, Oa, Ob, c = tasks[i]
            b = i % 2
            sl = pl.ds(base + c * C, C)
            ha = pltpu.async_copy(rows[b], Oa.at[sl], wsem[b])
            hb = pltpu.async_copy(rows[b], Ob.at[sl], wsem[b])
            return (ha, hb)

        n = len(tasks)
        g = [None, None]
        w = [None, None]
        g[0] = start_gather(0)
        for i in range(n):
            b = i % 2
            if i + 1 < n:
                nb = (i + 1) % 2
                if w[nb] is not None:
                    for h in w[nb]:
                        h.wait()
                    w[nb] = None
                g[nb] = start_gather(i + 1)
            g[b].wait()
            w[b] = start_writes(i)
        for pair in w:
            if pair is not None:
                for h in pair:
                    h.wait()

    return k(idx, t0, t1, t2)


def kernel(inputs, table0, table1, table2):
    B, S = inputs.shape
    idx = inputs.reshape(NW, NCHUNK, C).astype(jnp.int32)
    outs = _gather6(idx, table0, table1, table2)
    return tuple(o.reshape(B, S, D) for o in outs)
